# baseline (speedup 1.0000x reference)
import jax
import jax.numpy as jnp
from jax import lax
from jax.experimental import pallas as pl
from jax.experimental.pallas import tpu as pltpu
from jax.experimental.pallas import tpu_sc as plsc

MAXLEN = 200
MAXLEN_PAD = 256
EMBED_DIM = 64
PAD_DIM = 128
BATCH = 4096
LANES = 16
NC, NS = 2, 16
NW = NC * NS
ROWS_PER_W = BATCH // NW
SPLIT = 104
NVREG = EMBED_DIM // LANES
NBUF = 4
CHUNKS = ((0, SPLIT), (SPLIT, MAXLEN - SPLIT))


def _body(x_hbm, tab_hbm, out_hbm, idx_v, rows_v, gsems, wsems):
    wid = lax.axis_index("s") * NC + lax.axis_index("c")
    base = wid * ROWS_PER_W

    pltpu.sync_copy(x_hbm.at[pl.ds(base, ROWS_PER_W)], idx_v)

    def start_gather(r, b):
        for off, ln in CHUNKS:
            pltpu.async_copy(
                tab_hbm.at[idx_v.at[r, pl.ds(off, ln)]],
                rows_v.at[b, pl.ds(off, ln)],
                gsems.at[b],
            )

    def wait_gather(b):
        for off, ln in CHUNKS:
            pltpu.make_async_copy(
                tab_hbm.at[idx_v.at[0, pl.ds(off, ln)]],
                rows_v.at[b, pl.ds(off, ln)],
                gsems.at[b],
            ).wait()

    def issue_write(r, b):
        pltpu.async_copy(
            rows_v.at[b],
            out_hbm.at[base + r, :, pl.ds(0, EMBED_DIM)],
            wsems.at[b],
        )

    def wait_write(b):
        pltpu.make_async_copy(
            rows_v.at[b],
            out_hbm.at[base, :, pl.ds(0, EMBED_DIM)],
            wsems.at[b],
        ).wait()

    start_gather(0, 0)
    start_gather(1, 1)

    def outer(g, carry):
        for k in range(NBUF):
            r = g + k
            b = k
            wait_gather(b)
            issue_write(r, b)
            b2 = (k + 2) % NBUF

            @pl.when(r >= 2)
            def _():
                wait_write(b2)

            @pl.when(r + 2 < ROWS_PER_W)
            def _():
                start_gather(r + 2, b2)

        return carry

    lax.fori_loop(0, ROWS_PER_W // NBUF, lambda i, c: outer(i * NBUF, c), 0)

    wait_write(2)
    wait_write(3)


_emb = pl.kernel(
    _body,
    out_type=jax.ShapeDtypeStruct((BATCH, MAXLEN, PAD_DIM), jnp.float32),
    mesh=plsc.VectorSubcoreMesh(
        core_axis_name="c", subcore_axis_name="s", num_cores=NC, num_subcores=NS
    ),
    scratch_types=[
        pltpu.VMEM((ROWS_PER_W, MAXLEN_PAD), jnp.int32),
        pltpu.VMEM((NBUF, MAXLEN, EMBED_DIM), jnp.float32),
        pltpu.SemaphoreType.DMA((NBUF,)),
        pltpu.SemaphoreType.DMA((NBUF,)),
    ],
    compiler_params=pltpu.CompilerParams(use_tc_tiling_on_sc=False),
)


@jax.jit
def kernel(x, token_table, pos_table):
    x32 = jnp.pad(x.astype(jnp.int32), ((0, 0), (0, MAXLEN_PAD - MAXLEN)))
    y = _emb(x32, token_table)
    return y[:, :, :EMBED_DIM] + pos_table[None, :, :]


# speedup vs baseline: 1.5817x; 1.5817x over previous
import jax
import jax.numpy as jnp
from jax import lax
from jax.experimental import pallas as pl
from jax.experimental.pallas import tpu as pltpu
from jax.experimental.pallas import tpu_sc as plsc

MAXLEN = 200
MAXLEN_PAD = 256
EMBED_DIM = 64
PAD_DIM = 128
BATCH = 4096
LANES = 16
NC, NS = 2, 16
NW = NC * NS
ROWS_PER_W = BATCH // NW
SPLIT = 104
NVREG = EMBED_DIM // LANES
NBUF = 4
CHUNKS = ((0, SPLIT), (SPLIT, MAXLEN - SPLIT))


def _body(x_hbm, tab_hbm, pos_hbm, out_hbm, idx_v, rows_v, pos_v, gsems, wsems):
    wid = lax.axis_index("s") * NC + lax.axis_index("c")
    base = wid * ROWS_PER_W

    pltpu.sync_copy(pos_hbm, pos_v)
    pltpu.sync_copy(x_hbm.at[pl.ds(base, ROWS_PER_W)], idx_v)

    def start_gather(r, b):
        for off, ln in CHUNKS:
            pltpu.async_copy(
                tab_hbm.at[idx_v.at[r, pl.ds(off, ln)]],
                rows_v.at[b, pl.ds(off, ln)],
                gsems.at[b],
            )

    def wait_gather(b):
        for off, ln in CHUNKS:
            pltpu.make_async_copy(
                tab_hbm.at[idx_v.at[0, pl.ds(off, ln)]],
                rows_v.at[b, pl.ds(off, ln)],
                gsems.at[b],
            ).wait()

    def add_pos(b):
        def step(t, carry):
            for j in range(NVREG):
                plsc.addupdate(
                    rows_v.at[b, t, pl.ds(j * LANES, LANES)],
                    pos_v[t, pl.ds(j * LANES, LANES)],
                )
            return carry

        lax.fori_loop(0, MAXLEN, step, 0, unroll=4)

    def issue_write(r, b):
        pltpu.async_copy(
            rows_v.at[b],
            out_hbm.at[base + r, :, pl.ds(0, EMBED_DIM)],
            wsems.at[b],
        )

    def wait_write(b):
        pltpu.make_async_copy(
            rows_v.at[b],
            out_hbm.at[base, :, pl.ds(0, EMBED_DIM)],
            wsems.at[b],
        ).wait()

    start_gather(0, 0)
    start_gather(1, 1)

    def outer(g, carry):
        for k in range(NBUF):
            r = g + k
            b = k
            wait_gather(b)
            add_pos(b)
            issue_write(r, b)
            b2 = (k + 2) % NBUF

            @pl.when(r >= 2)
            def _():
                wait_write(b2)

            @pl.when(r + 2 < ROWS_PER_W)
            def _():
                start_gather(r + 2, b2)

        return carry

    lax.fori_loop(0, ROWS_PER_W // NBUF, lambda i, c: outer(i * NBUF, c), 0)

    wait_write(2)
    wait_write(3)


_emb = pl.kernel(
    _body,
    out_type=jax.ShapeDtypeStruct((BATCH, MAXLEN, PAD_DIM), jnp.float32),
    mesh=plsc.VectorSubcoreMesh(
        core_axis_name="c", subcore_axis_name="s", num_cores=NC, num_subcores=NS
    ),
    scratch_types=[
        pltpu.VMEM((ROWS_PER_W, MAXLEN), jnp.int32),
        pltpu.VMEM((NBUF, MAXLEN, EMBED_DIM), jnp.float32),
        pltpu.VMEM((MAXLEN, EMBED_DIM), jnp.float32),
        pltpu.SemaphoreType.DMA((NBUF,)),
        pltpu.SemaphoreType.DMA((NBUF,)),
    ],
    compiler_params=pltpu.CompilerParams(use_tc_tiling_on_sc=False),
)


@jax.jit
def kernel(x, token_table, pos_table):
    x32 = x.astype(jnp.int32)
    y = _emb(x32, token_table, pos_table)
    return y[:, :, :EMBED_DIM]


# speedup vs baseline: 1.5824x; 1.0004x over previous
import jax
import jax.numpy as jnp
from jax import lax
from jax.experimental import pallas as pl
from jax.experimental.pallas import tpu as pltpu
from jax.experimental.pallas import tpu_sc as plsc

MAXLEN = 200
MAXLEN_PAD = 256
EMBED_DIM = 64
PAD_DIM = 128
BATCH = 4096
LANES = 16
NC, NS = 2, 16
NW = NC * NS
ROWS_PER_W = BATCH // NW
SPLIT = 104
NVREG = EMBED_DIM // LANES
NBUF = 4
CHUNKS = ((0, SPLIT), (SPLIT, MAXLEN - SPLIT))


def _body(x_hbm, tab_hbm, pos_hbm, out_hbm, idx_v, rows_v, pos_v, gsems, wsems):
    wid = lax.axis_index("s") * NC + lax.axis_index("c")
    base = wid * ROWS_PER_W

    pltpu.sync_copy(pos_hbm, pos_v)
    pltpu.sync_copy(x_hbm.at[pl.ds(base, ROWS_PER_W)], idx_v)

    def start_gather(r, b):
        for off, ln in CHUNKS:
            pltpu.async_copy(
                tab_hbm.at[idx_v.at[r, pl.ds(off, ln)]],
                rows_v.at[b, pl.ds(off, ln)],
                gsems.at[b],
            )

    def wait_gather(b):
        for off, ln in CHUNKS:
            pltpu.make_async_copy(
                tab_hbm.at[idx_v.at[0, pl.ds(off, ln)]],
                rows_v.at[b, pl.ds(off, ln)],
                gsems.at[b],
            ).wait()

    def add_pos(b):
        def step(t, carry):
            for j in range(NVREG):
                plsc.addupdate(
                    rows_v.at[b, t, pl.ds(j * LANES, LANES)],
                    pos_v[t, pl.ds(j * LANES, LANES)],
                )
            return carry

        lax.fori_loop(0, MAXLEN, step, 0, unroll=8)

    def issue_write(r, b):
        pltpu.async_copy(
            rows_v.at[b],
            out_hbm.at[base + r, :, pl.ds(0, EMBED_DIM)],
            wsems.at[b],
        )

    def wait_write(b):
        pltpu.make_async_copy(
            rows_v.at[b],
            out_hbm.at[base, :, pl.ds(0, EMBED_DIM)],
            wsems.at[b],
        ).wait()

    start_gather(0, 0)
    start_gather(1, 1)

    def outer(g, carry):
        for k in range(NBUF):
            r = g + k
            b = k
            wait_gather(b)
            add_pos(b)
            issue_write(r, b)
            b2 = (k + 2) % NBUF

            @pl.when(r >= 2)
            def _():
                wait_write(b2)

            @pl.when(r + 2 < ROWS_PER_W)
            def _():
                start_gather(r + 2, b2)

        return carry

    lax.fori_loop(0, ROWS_PER_W // NBUF, lambda i, c: outer(i * NBUF, c), 0)

    wait_write(2)
    wait_write(3)


_emb = pl.kernel(
    _body,
    out_type=jax.ShapeDtypeStruct((BATCH, MAXLEN, PAD_DIM), jnp.float32),
    mesh=plsc.VectorSubcoreMesh(
        core_axis_name="c", subcore_axis_name="s", num_cores=NC, num_subcores=NS
    ),
    scratch_types=[
        pltpu.VMEM((ROWS_PER_W, MAXLEN), jnp.int32),
        pltpu.VMEM((NBUF, MAXLEN, EMBED_DIM), jnp.float32),
        pltpu.VMEM((MAXLEN, EMBED_DIM), jnp.float32),
        pltpu.SemaphoreType.DMA((NBUF,)),
        pltpu.SemaphoreType.DMA((NBUF,)),
    ],
    compiler_params=pltpu.CompilerParams(use_tc_tiling_on_sc=False),
)


@jax.jit
def kernel(x, token_table, pos_table):
    x32 = x.astype(jnp.int32)
    y = _emb(x32, token_table, pos_table)
    return y[:, :, :EMBED_DIM]


# speedup vs baseline: 1.6378x; 1.0350x over previous
import jax
import jax.numpy as jnp
from jax import lax
from jax.experimental import pallas as pl
from jax.experimental.pallas import tpu as pltpu
from jax.experimental.pallas import tpu_sc as plsc

MAXLEN = 200
MAXLEN_PAD = 256
EMBED_DIM = 64
PAD_DIM = 128
BATCH = 4096
LANES = 16
NC, NS = 2, 16
NW = NC * NS
ROWS_PER_W = BATCH // NW
SPLIT = 104
NVREG = EMBED_DIM // LANES
NBUF = 4
CHUNKS = ((0, SPLIT), (SPLIT, MAXLEN - SPLIT))


def _body(x_hbm, tab_hbm, pos_hbm, out_hbm, idx_v, rows_v, pos_v, gsems, wsems):
    wid = lax.axis_index("s") * NC + lax.axis_index("c")
    base = wid * ROWS_PER_W

    pltpu.sync_copy(pos_hbm, pos_v)
    pltpu.sync_copy(x_hbm.at[pl.ds(base, ROWS_PER_W)], idx_v)

    def start_gather(r, b):
        for off, ln in CHUNKS:
            pltpu.async_copy(
                tab_hbm.at[idx_v.at[r, pl.ds(off, ln)]],
                rows_v.at[b, pl.ds(off, ln)],
                gsems.at[b],
            )

    def wait_gather(b):
        for off, ln in CHUNKS:
            pltpu.make_async_copy(
                tab_hbm.at[idx_v.at[0, pl.ds(off, ln)]],
                rows_v.at[b, pl.ds(off, ln)],
                gsems.at[b],
            ).wait()

    def add_pos(b):
        def step(t, carry):
            for j in range(NVREG):
                plsc.addupdate(
                    rows_v.at[b, t, pl.ds(j * LANES, LANES)],
                    pos_v[t, pl.ds(j * LANES, LANES)],
                )
            return carry

        lax.fori_loop(0, MAXLEN, step, 0, unroll=8)

    def issue_write(r, b):
        pltpu.async_copy(
            rows_v.at[b],
            out_hbm.at[base + r, :, pl.ds(0, EMBED_DIM)],
            wsems.at[b],
        )

    def wait_write(b):
        pltpu.make_async_copy(
            rows_v.at[b],
            out_hbm.at[base, :, pl.ds(0, EMBED_DIM)],
            wsems.at[b],
        ).wait()

    start_gather(0, 0)
    start_gather(1, 1)
    start_gather(2, 2)

    def outer(g, carry):
        for k in range(NBUF):
            r = g + k
            b = k
            wait_gather(b)
            add_pos(b)
            issue_write(r, b)
            b3 = (k + 3) % NBUF

            @pl.when(r >= 1)
            def _():
                wait_write(b3)

            @pl.when(r + 3 < ROWS_PER_W)
            def _():
                start_gather(r + 3, b3)

        return carry

    lax.fori_loop(0, ROWS_PER_W // NBUF, lambda i, c: outer(i * NBUF, c), 0)

    wait_write(3)


_emb = pl.kernel(
    _body,
    out_type=jax.ShapeDtypeStruct((BATCH, MAXLEN, PAD_DIM), jnp.float32),
    mesh=plsc.VectorSubcoreMesh(
        core_axis_name="c", subcore_axis_name="s", num_cores=NC, num_subcores=NS
    ),
    scratch_types=[
        pltpu.VMEM((ROWS_PER_W, MAXLEN), jnp.int32),
        pltpu.VMEM((NBUF, MAXLEN, EMBED_DIM), jnp.float32),
        pltpu.VMEM((MAXLEN, EMBED_DIM), jnp.float32),
        pltpu.SemaphoreType.DMA((NBUF,)),
        pltpu.SemaphoreType.DMA((NBUF,)),
    ],
    compiler_params=pltpu.CompilerParams(use_tc_tiling_on_sc=False),
)


@jax.jit
def kernel(x, token_table, pos_table):
    x32 = x.astype(jnp.int32)
    y = _emb(x32, token_table, pos_table)
    return y[:, :, :EMBED_DIM]
